# baseline (speedup 1.0000x reference)
import jax
import jax.numpy as jnp
from jax.experimental import pallas as pl
from jax.experimental.pallas import tpu as pltpu

B, L, D = 16, 4096, 256
H, O = 512, 128
CHUNK = 1024
NC = L // CHUNK
NACC = 8


def _num_chunks(length):
    return (length + CHUNK - 1) // CHUNK


def _chunksum(load):
    m = CHUNK // NACC
    parts = []
    for a in range(NACC):
        s = load(a * m)
        for k in range(1, m // 8):
            s = s + load(a * m + k * 8)
        parts.append(s)
    while len(parts) > 1:
        parts = [parts[p] + parts[p + 1] for p in range(0, len(parts), 2)]
    return parts[0]


def _pool_kernel(l1_ref, l2_ref,
                 x1_ref, x2_ref,
                 e1_ref, e2_ref, acc1_ref, acc2_ref):
    i = pl.program_id(0)
    j = pl.program_id(1)
    base = j * CHUNK

    def accum(len_ref, x_ref, acc_ref, e_ref):
        length = len_ref[i]

        @pl.when(j == 0)
        def _():
            acc_ref[...] = jnp.zeros_like(acc_ref)

        @pl.when(base + CHUNK <= length)
        def _():
            acc_ref[...] += _chunksum(
                lambda lo: x_ref[0, pl.ds(lo, 8), :])

        @pl.when((base < length) & (length < base + CHUNK))
        def _():
            lim = length - base
            iota8 = jax.lax.broadcasted_iota(jnp.int32, (8, 1), 0)

            def load_masked(lo):
                return jnp.where(iota8 + lo < lim,
                                 x_ref[0, pl.ds(lo, 8), :], 0.0)

            acc_ref[...] += _chunksum(load_masked)

        @pl.when(j == NC - 1)
        def _():
            e_ref[0] = jnp.sum(acc_ref[...], axis=0, keepdims=True)

    accum(l1_ref, x1_ref, acc1_ref, e1_ref)
    accum(l2_ref, x2_ref, acc2_ref, e2_ref)


def _mlp_kernel(e1_ref, e2_ref, len1f_ref, len2f_ref,
                w1_ref, b1_ref, w2_ref, b2_ref, out_ref):
    e1 = e1_ref[...] / len1f_ref[...]
    e2 = e2_ref[...] / len2f_ref[...]
    cat = jnp.concatenate([e1, e2, jnp.abs(e1 - e2), e1 * e2], axis=1)
    h = jnp.dot(cat, w1_ref[...], preferred_element_type=jnp.float32)
    h = jnp.maximum(h + b1_ref[...], 0.0)
    out_ref[...] = (
        jnp.dot(h, w2_ref[...], preferred_element_type=jnp.float32)
        + b2_ref[...]
    )


def kernel(X1, x1_lengths, X2, x2_lengths, W1, b1, W2, b2):
    def x_spec(which):
        def index_map(i, j, l1, l2):
            lens = l1 if which == 0 else l2
            return (i, jnp.minimum(j, _num_chunks(lens[i]) - 1), 0)
        return pl.BlockSpec((1, CHUNK, D), index_map)

    pool_spec = pltpu.PrefetchScalarGridSpec(
        num_scalar_prefetch=2,
        grid=(B, NC),
        in_specs=[x_spec(0), x_spec(1)],
        out_specs=[
            pl.BlockSpec((1, 1, D), lambda i, j, l1, l2: (i, 0, 0)),
            pl.BlockSpec((1, 1, D), lambda i, j, l1, l2: (i, 0, 0)),
        ],
        scratch_shapes=[
            pltpu.VMEM((8, D), jnp.float32),
            pltpu.VMEM((8, D), jnp.float32),
        ],
    )

    e1s, e2s = pl.pallas_call(
        _pool_kernel,
        grid_spec=pool_spec,
        out_shape=[
            jax.ShapeDtypeStruct((B, 1, D), jnp.float32),
            jax.ShapeDtypeStruct((B, 1, D), jnp.float32),
        ],
        compiler_params=pltpu.CompilerParams(
            dimension_semantics=("arbitrary", "arbitrary"),
        ),
    )(x1_lengths, x2_lengths, X1, X2)

    len1f = x1_lengths.astype(jnp.float32).reshape(B, 1)
    len2f = x2_lengths.astype(jnp.float32).reshape(B, 1)
    return pl.pallas_call(
        _mlp_kernel,
        out_shape=jax.ShapeDtypeStruct((B, O), jnp.float32),
    )(e1s.reshape(B, D), e2s.reshape(B, D), len1f, len2f,
      W1, b1.reshape(1, H), W2, b2.reshape(1, O))
